# baseline (speedup 1.0000x reference)
import functools

import numpy as np
import jax
import jax.numpy as jnp
from jax import lax
from jax.experimental import pallas as pl
from jax.experimental.pallas import tpu as pltpu

_LANE = 128
_SUBLANE = 8


def _gem_dense_kernel(x_ref, s_ref, o_ref, *, eps, inv_hw, inv_p):
    x = jnp.maximum(x_ref[...], jnp.float32(eps))
    xp = (x * x * x).astype(jnp.bfloat16)
    acc = jnp.dot(xp, s_ref[...], preferred_element_type=jnp.float32)
    o_ref[...] = jnp.power(acc * jnp.float32(inv_hw), jnp.float32(inv_p))


@functools.lru_cache(maxsize=None)
def _selection_matrix(hw: int, g: int):
    sel = np.kron(np.eye(g, dtype=np.float32), np.ones((hw, 1), np.float32))
    return jnp.asarray(sel, dtype=jnp.bfloat16)


def _gem(x, p=3.0, eps=1e-6):
    N, C, H, W = x.shape
    R, HW = N * C, H * W
    g = int(np.lcm(HW, _LANE)) // HW
    assert R % g == 0, (R, g)
    dr, dw = R // g, g * HW
    xd = x.reshape(dr, dw)

    tr = 64
    while dr % tr != 0:
        tr //= 2
    grid = dr // tr

    kernel_fn = functools.partial(
        _gem_dense_kernel, eps=float(eps), inv_hw=1.0 / float(HW),
        inv_p=1.0 / float(p))
    out = pl.pallas_call(
        kernel_fn,
        out_shape=jax.ShapeDtypeStruct((dr, g), x.dtype),
        grid=(grid,),
        in_specs=[
            pl.BlockSpec((tr, dw), lambda i: (i, 0)),
            pl.BlockSpec((dw, g), lambda i: (0, 0)),
        ],
        out_specs=pl.BlockSpec((tr, g), lambda i: (i, 0)),
        compiler_params=pltpu.CompilerParams(
            dimension_semantics=("parallel",),
            vmem_limit_bytes=int(32 << 20)),
    )(xd, _selection_matrix(HW, g))
    return out.reshape(N, C, 1, 1)


def kernel(x):
    return _gem(x, p=3.0, eps=1e-6)


# speedup vs baseline: 26.1174x; 26.1174x over previous
import functools

import jax
import jax.numpy as jnp
from jax.experimental import pallas as pl
from jax.experimental.pallas import tpu as pltpu


def _gem_planes_kernel(x_ref, o_ref, *, hw, eps, inv_hw, inv_p):
    def body(i, acc):
        x = jnp.maximum(x_ref[i], jnp.float32(eps))
        return acc + x * x * x
    acc = jax.lax.fori_loop(
        0, hw, body, jnp.zeros(o_ref.shape, jnp.float32), unroll=True)
    o_ref[...] = jnp.power(acc * jnp.float32(inv_hw), jnp.float32(inv_p))


def _gem(x, p=3.0, eps=1e-6):
    N, C, H, W = x.shape
    HW = H * W
    xt = jnp.transpose(x, (2, 3, 0, 1)).reshape(HW, N, C)

    bc = 256
    while C % bc != 0:
        bc //= 2
    grid = C // bc

    kernel_fn = functools.partial(
        _gem_planes_kernel, hw=HW, eps=float(eps), inv_hw=1.0 / float(HW),
        inv_p=1.0 / float(p))
    out = pl.pallas_call(
        kernel_fn,
        out_shape=jax.ShapeDtypeStruct((N, C), x.dtype),
        grid=(grid,),
        in_specs=[pl.BlockSpec((HW, N, bc), lambda j: (0, 0, j))],
        out_specs=pl.BlockSpec((N, bc), lambda j: (0, j)),
        compiler_params=pltpu.CompilerParams(
            dimension_semantics=("parallel",),
            vmem_limit_bytes=int(32 << 20)),
    )(xt)
    return out.reshape(N, C, 1, 1)


def kernel(x):
    return _gem(x, p=3.0, eps=1e-6)


# speedup vs baseline: 29.2891x; 1.1214x over previous
import functools

import jax
import jax.numpy as jnp
from jax.experimental import pallas as pl
from jax.experimental.pallas import tpu as pltpu


def _gem_planes_kernel(x_ref, o_ref, *, hw, eps, inv_hw, inv_p):
    def body(i, acc):
        x = jnp.maximum(x_ref[i], jnp.float32(eps))
        return acc + x * x * x
    acc = jax.lax.fori_loop(
        0, hw, body, jnp.zeros(o_ref.shape, jnp.float32), unroll=True)
    o_ref[...] = jnp.power(acc * jnp.float32(inv_hw), jnp.float32(inv_p))


def _gem(x, p=3.0, eps=1e-6):
    N, C, H, W = x.shape
    HW = H * W
    xt = jnp.transpose(x, (2, 3, 0, 1)).reshape(HW, N, C)

    bc = 512
    while C % bc != 0:
        bc //= 2
    grid = C // bc

    kernel_fn = functools.partial(
        _gem_planes_kernel, hw=HW, eps=float(eps), inv_hw=1.0 / float(HW),
        inv_p=1.0 / float(p))
    out = pl.pallas_call(
        kernel_fn,
        out_shape=jax.ShapeDtypeStruct((N, C), x.dtype),
        grid=(grid,),
        in_specs=[pl.BlockSpec((HW, N, bc), lambda j: (0, 0, j))],
        out_specs=pl.BlockSpec((N, bc), lambda j: (0, j)),
        compiler_params=pltpu.CompilerParams(
            dimension_semantics=("parallel",),
            vmem_limit_bytes=int(32 << 20)),
    )(xt)
    return out.reshape(N, C, 1, 1)


def kernel(x):
    return _gem(x, p=3.0, eps=1e-6)
